# baseline (speedup 1.0000x reference)
import functools

import jax
import jax.numpy as jnp
from jax import lax
from jax.experimental import pallas as pl
from jax.experimental.pallas import tpu as pltpu
from jax.experimental.pallas import tpu_sc as plsc

N = 10000
D = 128
H2 = 256
E = 320000
N_CLASSES = 10

NC = 2
NS = 16
NW = NC * NS
K = 128
CH = 80
HCH = CH // 2
T = CH * K
EP = NW * T
R = 10112
DUMP = N
RPT = R // NS
DEGW = 16
BLK = 1264
GRID = R // BLK


def _mesh():
  return plsc.VectorSubcoreMesh(core_axis_name="c", subcore_axis_name="s",
                                num_cores=NC, num_subcores=NS)


def _segsum_body(h, srcs, dsts, zrows, out,
                 src_v, dst_v, rows0, rows1, acc, sem0, sem1):
  cid = lax.axis_index("c")
  sid = lax.axis_index("s")
  wid = sid * NC + cid
  base = sid * RPT

  pltpu.sync_copy(zrows, acc.at[pl.ds(base, RPT)])
  plsc.subcore_barrier()

  for half in range(2):
    pltpu.sync_copy(srcs.at[pl.ds(wid * CH + half * HCH, HCH)], src_v)
    pltpu.sync_copy(dsts.at[pl.ds(wid * CH + half * HCH, HCH)], dst_v)
    pltpu.async_copy(h.at[src_v.at[0]], rows0, sem0)

    def step(i, carry):
      c0 = 2 * i
      c1 = c0 + 1
      pltpu.async_copy(h.at[src_v.at[c1]], rows1, sem1)
      pltpu.make_async_copy(h.at[src_v.at[c0]], rows0, sem0).wait()
      pltpu.sync_copy(rows0, acc.at[dst_v.at[c0]], add=True)

      @pl.when(i < HCH // 2 - 1)
      def _():
        pltpu.async_copy(h.at[src_v.at[c0 + 2]], rows0, sem0)

      pltpu.make_async_copy(h.at[src_v.at[c1]], rows1, sem1).wait()
      pltpu.sync_copy(rows1, acc.at[dst_v.at[c1]], add=True)
      return carry

    lax.fori_loop(0, HCH // 2, step, 0)
  plsc.subcore_barrier()

  pltpu.sync_copy(acc.at[pl.ds(base, RPT)],
                  out.at[pl.ds(cid * R + base, RPT)])


def _make_segsum():
  scratch = (
      pltpu.VMEM((HCH, K), jnp.int32),
      pltpu.VMEM((HCH, K), jnp.int32),
      pltpu.VMEM((K, D), jnp.float32),
      pltpu.VMEM((K, D), jnp.float32),
      pltpu.VMEM_SHARED((R, D), jnp.float32),
      pltpu.SemaphoreType.DMA,
      pltpu.SemaphoreType.DMA,
  )
  return pl.kernel(_segsum_body,
                   out_type=jax.ShapeDtypeStruct((NC * R, D), jnp.float32),
                   mesh=_mesh(), scratch_types=scratch)


def _deg_body(dsts, ones128, zrows, degout, dst_v, ones_v, dacc):
  cid = lax.axis_index("c")
  sid = lax.axis_index("s")
  wid = sid * NC + cid
  base = sid * RPT

  pltpu.sync_copy(dsts.at[pl.ds(wid * CH, CH)], dst_v)
  pltpu.sync_copy(zrows, dacc.at[pl.ds(base, RPT)])
  pltpu.sync_copy(ones128, ones_v)
  plsc.subcore_barrier()

  def step(c, carry):
    pltpu.sync_copy(ones_v, dacc.at[dst_v.at[c]], add=True)
    return carry

  lax.fori_loop(0, CH, step, 0)
  plsc.subcore_barrier()
  pltpu.sync_copy(dacc.at[pl.ds(base, RPT)],
                  degout.at[pl.ds(cid * R + base, RPT)])


def _make_deg():
  scratch = (
      pltpu.VMEM((CH, K), jnp.int32),
      pltpu.VMEM((K, D), jnp.float32),
      pltpu.VMEM_SHARED((R, D), jnp.float32),
  )
  return pl.kernel(_deg_body,
                   out_type=jax.ShapeDtypeStruct((NC * R, D), jnp.float32),
                   mesh=_mesh(), scratch_types=scratch)


def _mlp_body(h_ref, p_ref, dg_ref, w1_ref, b1_ref, w2_ref, b2_ref,
              s_ref, t_ref, o_ref):
  h = h_ref[...]
  agg = (p_ref[0] + p_ref[1]) / jnp.maximum(dg_ref[...], 1.0)
  m = h + agg
  x = jnp.dot(m, w1_ref[...], preferred_element_type=jnp.float32,
              precision=lax.Precision.HIGHEST)
  x = jnp.maximum(x + b1_ref[...], 0.0)
  y = jnp.dot(x, w2_ref[...], preferred_element_type=jnp.float32,
              precision=lax.Precision.HIGHEST)
  y = y + b2_ref[...] + h
  o_ref[...] = y * s_ref[...] + t_ref[...]


def _mlp_call(h, p3, deg128, w1, b1, w2, b2, s, t):
  return pl.pallas_call(
      _mlp_body,
      grid=(GRID,),
      in_specs=[
          pl.BlockSpec((BLK, D), lambda j: (j, 0)),
          pl.BlockSpec((2, BLK, D), lambda j: (0, j, 0)),
          pl.BlockSpec((BLK, D), lambda j: (j, 0)),
          pl.BlockSpec((D, H2), lambda j: (0, 0)),
          pl.BlockSpec((1, H2), lambda j: (0, 0)),
          pl.BlockSpec((H2, D), lambda j: (0, 0)),
          pl.BlockSpec((1, D), lambda j: (0, 0)),
          pl.BlockSpec((1, D), lambda j: (0, 0)),
          pl.BlockSpec((1, D), lambda j: (0, 0)),
      ],
      out_specs=pl.BlockSpec((BLK, D), lambda j: (j, 0)),
      out_shape=jax.ShapeDtypeStruct((R, D), jnp.float32),
  )(h, p3, deg128, w1, b1, w2, b2, s, t)


def _readout_body(h_ref, w0_ref, b0_ref, w1_ref, b1_ref, w2_ref, b2_ref,
                  o_ref):
  h = h_ref[...]
  rows = lax.broadcasted_iota(jnp.int32, (R, 1), 0)
  g = jnp.sum(jnp.where(rows < N, h, 0.0), axis=0, keepdims=True) * (1.0 / N)
  g8 = jnp.broadcast_to(g, (8, D))
  x = jnp.maximum(jnp.dot(g8, w0_ref[...], preferred_element_type=jnp.float32,
                          precision=lax.Precision.HIGHEST) + b0_ref[...], 0.0)
  x = jnp.maximum(jnp.dot(x, w1_ref[...], preferred_element_type=jnp.float32,
                          precision=lax.Precision.HIGHEST) + b1_ref[...], 0.0)
  y = jnp.dot(x, w2_ref[...], preferred_element_type=jnp.float32,
              precision=lax.Precision.HIGHEST) + b2_ref[...]
  o_ref[...] = y


def _readout_call(h, w0, b0, w1, b1, w2, b2):
  return pl.pallas_call(
      _readout_body,
      out_shape=jax.ShapeDtypeStruct((8, D), jnp.float32),
  )(h, w0, b0, w1, b1, w2, b2)


def kernel(node_feat, edge_index, edge_feat, snorm_n, snorm_e, params):
  del edge_feat, snorm_n, snorm_e
  f32 = jnp.float32

  h = jnp.pad(node_feat, ((0, R - N), (0, 0)))
  perm = jnp.argsort(edge_index[0])
  src = edge_index[0][perm]
  dst = edge_index[1][perm]
  srcs = jnp.concatenate(
      [src, jnp.zeros((EP - E,), jnp.int32)]).reshape(NW * CH, K)
  dsts = jnp.concatenate(
      [dst, jnp.full((EP - E,), DUMP, jnp.int32)]).reshape(NW * CH, K)
  zrows = jnp.zeros((RPT, D), f32)
  ones128 = jnp.ones((K, D), f32)

  segsum = _make_segsum()
  deg_kernel = _make_deg()

  degp = deg_kernel(dsts, ones128, zrows)
  deg128 = degp[:R] + degp[R:]

  for i in range(3):
    lp = params["layers"][i]
    bp = params["bns"][i]
    s = (bp["g"] / jnp.sqrt(bp["rv"] + 1e-5)).reshape(1, D)
    t = (bp["bta"] - bp["rm"] * s[0]).reshape(1, D)
    part = segsum(h, srcs, dsts, zrows)
    h = _mlp_call(h, part.reshape(NC, R, D), deg128,
                  lp["W1"], lp["b1"].reshape(1, H2),
                  lp["W2"], lp["b2"].reshape(1, D), s, t)

  ro = params["readout"]
  w0 = jnp.zeros((D, D), f32).at[:, :64].set(ro[0]["W"])
  b0 = jnp.zeros((1, D), f32).at[0, :64].set(ro[0]["b"])
  w1 = jnp.zeros((D, D), f32).at[:64, :32].set(ro[1]["W"])
  b1 = jnp.zeros((1, D), f32).at[0, :32].set(ro[1]["b"])
  w2 = jnp.zeros((D, D), f32).at[:32, :N_CLASSES].set(ro[2]["W"])
  b2 = jnp.zeros((1, D), f32).at[0, :N_CLASSES].set(ro[2]["b"])
  out = _readout_call(h, w0, b0, w1, b1, w2, b2)
  return out[:1, :N_CLASSES]


# speedup vs baseline: 1.3066x; 1.3066x over previous
import functools

import jax
import jax.numpy as jnp
from jax import lax
from jax.experimental import pallas as pl
from jax.experimental.pallas import tpu as pltpu
from jax.experimental.pallas import tpu_sc as plsc

N = 10000
D = 128
H2 = 256
E = 320000
N_CLASSES = 10

NC = 2
NS = 16
NW = NC * NS
K = 64
CH = 160
WIN = 40
NWIN = CH // WIN
NBUF = 4
T = CH * K
EP = NW * T
R = 10112
DUMP = N
RPT = R // NS
DEGW = 16
BLK = 1264
GRID = R // BLK


def _mesh():
  return plsc.VectorSubcoreMesh(core_axis_name="c", subcore_axis_name="s",
                                num_cores=NC, num_subcores=NS)


def _segsum_body(h, srcs, dsts, zrows, out, src_v, dst_v,
                 rows0, rows1, rows2, rows3, acc, s0, s1, s2, s3):
  cid = lax.axis_index("c")
  sid = lax.axis_index("s")
  wid = sid * NC + cid
  base = sid * RPT
  rows = (rows0, rows1, rows2, rows3)
  sems = (s0, s1, s2, s3)

  pltpu.sync_copy(zrows, acc.at[pl.ds(base, RPT)])
  plsc.subcore_barrier()

  for w in range(NWIN):
    pltpu.sync_copy(srcs.at[pl.ds(wid * CH + w * WIN, WIN)], src_v)
    pltpu.sync_copy(dsts.at[pl.ds(wid * CH + w * WIN, WIN)], dst_v)
    for b in range(NBUF):
      pltpu.async_copy(h.at[src_v.at[b]], rows[b], sems[b])

    def step(i, carry):
      for b in range(NBUF):
        c = i * NBUF + b
        pltpu.make_async_copy(h.at[src_v.at[c]], rows[b], sems[b]).wait()
        pltpu.sync_copy(rows[b], acc.at[dst_v.at[c]], add=True)

        @pl.when(i < WIN // NBUF - 1)
        def _():
          pltpu.async_copy(h.at[src_v.at[c + NBUF]], rows[b], sems[b])
      return carry

    lax.fori_loop(0, WIN // NBUF, step, 0)
  plsc.subcore_barrier()

  pltpu.sync_copy(acc.at[pl.ds(base, RPT)],
                  out.at[pl.ds(cid * R + base, RPT)])


def _make_segsum():
  scratch = (
      pltpu.VMEM((WIN, K), jnp.int32),
      pltpu.VMEM((WIN, K), jnp.int32),
      pltpu.VMEM((K, D), jnp.float32),
      pltpu.VMEM((K, D), jnp.float32),
      pltpu.VMEM((K, D), jnp.float32),
      pltpu.VMEM((K, D), jnp.float32),
      pltpu.VMEM_SHARED((R, D), jnp.float32),
      pltpu.SemaphoreType.DMA,
      pltpu.SemaphoreType.DMA,
      pltpu.SemaphoreType.DMA,
      pltpu.SemaphoreType.DMA,
  )
  return pl.kernel(_segsum_body,
                   out_type=jax.ShapeDtypeStruct((NC * R, D), jnp.float32),
                   mesh=_mesh(), scratch_types=scratch)


def _deg_body(dsts, ones128, zrows, degout, dst_v, ones_v, dacc):
  cid = lax.axis_index("c")
  sid = lax.axis_index("s")
  wid = sid * NC + cid
  base = sid * RPT

  pltpu.sync_copy(dsts.at[pl.ds(wid * CH, CH)], dst_v)
  pltpu.sync_copy(zrows, dacc.at[pl.ds(base, RPT)])
  pltpu.sync_copy(ones128, ones_v)
  plsc.subcore_barrier()

  def step(c, carry):
    pltpu.sync_copy(ones_v, dacc.at[dst_v.at[c]], add=True)
    return carry

  lax.fori_loop(0, CH, step, 0)
  plsc.subcore_barrier()
  pltpu.sync_copy(dacc.at[pl.ds(base, RPT)],
                  degout.at[pl.ds(cid * R + base, RPT)])


def _make_deg():
  scratch = (
      pltpu.VMEM((CH, K), jnp.int32),
      pltpu.VMEM((K, D), jnp.float32),
      pltpu.VMEM_SHARED((R, D), jnp.float32),
  )
  return pl.kernel(_deg_body,
                   out_type=jax.ShapeDtypeStruct((NC * R, D), jnp.float32),
                   mesh=_mesh(), scratch_types=scratch)


def _mlp_body(h_ref, p_ref, dg_ref, w1_ref, b1_ref, w2_ref, b2_ref,
              s_ref, t_ref, o_ref):
  h = h_ref[...]
  agg = (p_ref[0] + p_ref[1]) / jnp.maximum(dg_ref[...], 1.0)
  m = h + agg
  x = jnp.dot(m, w1_ref[...], preferred_element_type=jnp.float32,
              precision=lax.Precision.HIGHEST)
  x = jnp.maximum(x + b1_ref[...], 0.0)
  y = jnp.dot(x, w2_ref[...], preferred_element_type=jnp.float32,
              precision=lax.Precision.HIGHEST)
  y = y + b2_ref[...] + h
  o_ref[...] = y * s_ref[...] + t_ref[...]


def _mlp_call(h, p3, deg128, w1, b1, w2, b2, s, t):
  return pl.pallas_call(
      _mlp_body,
      grid=(GRID,),
      in_specs=[
          pl.BlockSpec((BLK, D), lambda j: (j, 0)),
          pl.BlockSpec((2, BLK, D), lambda j: (0, j, 0)),
          pl.BlockSpec((BLK, D), lambda j: (j, 0)),
          pl.BlockSpec((D, H2), lambda j: (0, 0)),
          pl.BlockSpec((1, H2), lambda j: (0, 0)),
          pl.BlockSpec((H2, D), lambda j: (0, 0)),
          pl.BlockSpec((1, D), lambda j: (0, 0)),
          pl.BlockSpec((1, D), lambda j: (0, 0)),
          pl.BlockSpec((1, D), lambda j: (0, 0)),
      ],
      out_specs=pl.BlockSpec((BLK, D), lambda j: (j, 0)),
      out_shape=jax.ShapeDtypeStruct((R, D), jnp.float32),
  )(h, p3, deg128, w1, b1, w2, b2, s, t)


def _readout_body(h_ref, w0_ref, b0_ref, w1_ref, b1_ref, w2_ref, b2_ref,
                  o_ref):
  h = h_ref[...]
  rows = lax.broadcasted_iota(jnp.int32, (R, 1), 0)
  g = jnp.sum(jnp.where(rows < N, h, 0.0), axis=0, keepdims=True) * (1.0 / N)
  g8 = jnp.broadcast_to(g, (8, D))
  x = jnp.maximum(jnp.dot(g8, w0_ref[...], preferred_element_type=jnp.float32,
                          precision=lax.Precision.HIGHEST) + b0_ref[...], 0.0)
  x = jnp.maximum(jnp.dot(x, w1_ref[...], preferred_element_type=jnp.float32,
                          precision=lax.Precision.HIGHEST) + b1_ref[...], 0.0)
  y = jnp.dot(x, w2_ref[...], preferred_element_type=jnp.float32,
              precision=lax.Precision.HIGHEST) + b2_ref[...]
  o_ref[...] = y


def _readout_call(h, w0, b0, w1, b1, w2, b2):
  return pl.pallas_call(
      _readout_body,
      out_shape=jax.ShapeDtypeStruct((8, D), jnp.float32),
  )(h, w0, b0, w1, b1, w2, b2)


def kernel(node_feat, edge_index, edge_feat, snorm_n, snorm_e, params):
  del edge_feat, snorm_n, snorm_e
  f32 = jnp.float32

  h = jnp.pad(node_feat, ((0, R - N), (0, 0)))
  src = edge_index[0]
  dst = edge_index[1]
  srcs = jnp.concatenate(
      [src, jnp.zeros((EP - E,), jnp.int32)]).reshape(NW * CH, K)
  dsts = jnp.concatenate(
      [dst, jnp.full((EP - E,), DUMP, jnp.int32)]).reshape(NW * CH, K)
  zrows = jnp.zeros((RPT, D), f32)
  ones128 = jnp.ones((K, D), f32)

  segsum = _make_segsum()
  deg_kernel = _make_deg()

  degp = deg_kernel(dsts, ones128, zrows)
  deg128 = degp[:R] + degp[R:]

  for i in range(3):
    lp = params["layers"][i]
    bp = params["bns"][i]
    s = (bp["g"] / jnp.sqrt(bp["rv"] + 1e-5)).reshape(1, D)
    t = (bp["bta"] - bp["rm"] * s[0]).reshape(1, D)
    part = segsum(h, srcs, dsts, zrows)
    h = _mlp_call(h, part.reshape(NC, R, D), deg128,
                  lp["W1"], lp["b1"].reshape(1, H2),
                  lp["W2"], lp["b2"].reshape(1, D), s, t)

  ro = params["readout"]
  w0 = jnp.zeros((D, D), f32).at[:, :64].set(ro[0]["W"])
  b0 = jnp.zeros((1, D), f32).at[0, :64].set(ro[0]["b"])
  w1 = jnp.zeros((D, D), f32).at[:64, :32].set(ro[1]["W"])
  b1 = jnp.zeros((1, D), f32).at[0, :32].set(ro[1]["b"])
  w2 = jnp.zeros((D, D), f32).at[:32, :N_CLASSES].set(ro[2]["W"])
  b2 = jnp.zeros((1, D), f32).at[0, :N_CLASSES].set(ro[2]["b"])
  out = _readout_call(h, w0, b0, w1, b1, w2, b2)
  return out[:1, :N_CLASSES]


# speedup vs baseline: 1.3818x; 1.0576x over previous
import functools

import jax
import jax.numpy as jnp
from jax import lax
from jax.experimental import pallas as pl
from jax.experimental.pallas import tpu as pltpu
from jax.experimental.pallas import tpu_sc as plsc

N = 10000
D = 128
H2 = 256
E = 320000
N_CLASSES = 10

NC = 2
NS = 16
NW = NC * NS
K = 128
CH_A = 80
CH_B = 80
CH = CH_A + CH_B
WIN = 40
NBUF = 2
EP = NS * CH * K
R = 10112
DUMP = N
RPT = R // NS
DEGW = 16
BLK = 1264
GRID = R // BLK


def _mesh():
  return plsc.VectorSubcoreMesh(core_axis_name="c", subcore_axis_name="s",
                                num_cores=NC, num_subcores=NS)


def _segsum_body(h, srcs, dsts, zrows, out, src_v, dst_v,
                 rows0, rows1, acc, s0, s1):
  cid = lax.axis_index("c")
  sid = lax.axis_index("s")
  base = sid * RPT
  rows = (rows0, rows1)
  sems = (s0, s1)

  pltpu.sync_copy(zrows, acc.at[pl.ds(base, RPT)])
  plsc.subcore_barrier()

  def run(row_base, nch):
    for w in range(nch // WIN):
      pltpu.sync_copy(srcs.at[pl.ds(row_base + w * WIN, WIN)], src_v)
      pltpu.sync_copy(dsts.at[pl.ds(row_base + w * WIN, WIN)], dst_v)
      for b in range(NBUF):
        pltpu.async_copy(h.at[src_v.at[b]], rows[b], sems[b])

      def step(i, carry):
        for b in range(NBUF):
          c = i * NBUF + b
          pltpu.make_async_copy(h.at[src_v.at[c]], rows[b], sems[b]).wait()
          pltpu.sync_copy(rows[b], acc.at[dst_v.at[c]], add=True)

          @pl.when(i < WIN // NBUF - 1)
          def _():
            pltpu.async_copy(h.at[src_v.at[c + NBUF]], rows[b], sems[b])
        return carry

      lax.fori_loop(0, WIN // NBUF, step, 0)

  @pl.when(cid == 0)
  def _():
    run(sid * CH_A, CH_A)

  @pl.when(cid == 1)
  def _():
    run(NS * CH_A + sid * CH_B, CH_B)

  plsc.subcore_barrier()

  pltpu.sync_copy(acc.at[pl.ds(base, RPT)],
                  out.at[pl.ds(cid * R + base, RPT)])


def _make_segsum():
  scratch = (
      pltpu.VMEM((WIN, K), jnp.int32),
      pltpu.VMEM((WIN, K), jnp.int32),
      pltpu.VMEM((K, D), jnp.float32),
      pltpu.VMEM((K, D), jnp.float32),
      pltpu.VMEM_SHARED((R, D), jnp.float32),
      pltpu.SemaphoreType.DMA,
      pltpu.SemaphoreType.DMA,
  )
  return pl.kernel(_segsum_body,
                   out_type=jax.ShapeDtypeStruct((NC * R, D), jnp.float32),
                   mesh=_mesh(), scratch_types=scratch)


def _deg_body(dsts, ones128, zrows, degout, dst_v, ones_v, dacc):
  cid = lax.axis_index("c")
  sid = lax.axis_index("s")
  base = sid * RPT

  pltpu.sync_copy(zrows, dacc.at[pl.ds(base, RPT)])
  pltpu.sync_copy(ones128, ones_v)
  plsc.subcore_barrier()

  def run(row_base, nch):
    pltpu.sync_copy(dsts.at[pl.ds(row_base, nch)], dst_v.at[pl.ds(0, nch)])

    def step(c, carry):
      pltpu.sync_copy(ones_v, dacc.at[dst_v.at[c]], add=True)
      return carry

    lax.fori_loop(0, nch, step, 0)

  @pl.when(cid == 0)
  def _():
    run(sid * CH_A, CH_A)

  @pl.when(cid == 1)
  def _():
    run(NS * CH_A + sid * CH_B, CH_B)

  plsc.subcore_barrier()
  pltpu.sync_copy(dacc.at[pl.ds(base, RPT)],
                  degout.at[pl.ds(cid * R + base, RPT)])


def _make_deg():
  scratch = (
      pltpu.VMEM((max(CH_A, CH_B), K), jnp.int32),
      pltpu.VMEM((K, D), jnp.float32),
      pltpu.VMEM_SHARED((R, D), jnp.float32),
  )
  return pl.kernel(_deg_body,
                   out_type=jax.ShapeDtypeStruct((NC * R, D), jnp.float32),
                   mesh=_mesh(), scratch_types=scratch)


def _mlp_body(h_ref, p_ref, dg_ref, w1_ref, b1_ref, w2_ref, b2_ref,
              s_ref, t_ref, o_ref):
  h = h_ref[...]
  agg = (p_ref[0] + p_ref[1]) / jnp.maximum(dg_ref[...], 1.0)
  m = h + agg
  x = jnp.dot(m, w1_ref[...], preferred_element_type=jnp.float32,
              precision=lax.Precision.HIGHEST)
  x = jnp.maximum(x + b1_ref[...], 0.0)
  y = jnp.dot(x, w2_ref[...], preferred_element_type=jnp.float32,
              precision=lax.Precision.HIGHEST)
  y = y + b2_ref[...] + h
  o_ref[...] = y * s_ref[...] + t_ref[...]


def _mlp_call(h, p3, deg128, w1, b1, w2, b2, s, t):
  return pl.pallas_call(
      _mlp_body,
      grid=(GRID,),
      in_specs=[
          pl.BlockSpec((BLK, D), lambda j: (j, 0)),
          pl.BlockSpec((2, BLK, D), lambda j: (0, j, 0)),
          pl.BlockSpec((BLK, D), lambda j: (j, 0)),
          pl.BlockSpec((D, H2), lambda j: (0, 0)),
          pl.BlockSpec((1, H2), lambda j: (0, 0)),
          pl.BlockSpec((H2, D), lambda j: (0, 0)),
          pl.BlockSpec((1, D), lambda j: (0, 0)),
          pl.BlockSpec((1, D), lambda j: (0, 0)),
          pl.BlockSpec((1, D), lambda j: (0, 0)),
      ],
      out_specs=pl.BlockSpec((BLK, D), lambda j: (j, 0)),
      out_shape=jax.ShapeDtypeStruct((R, D), jnp.float32),
  )(h, p3, deg128, w1, b1, w2, b2, s, t)


def _readout_body(h_ref, w0_ref, b0_ref, w1_ref, b1_ref, w2_ref, b2_ref,
                  o_ref):
  h = h_ref[...]
  rows = lax.broadcasted_iota(jnp.int32, (R, 1), 0)
  g = jnp.sum(jnp.where(rows < N, h, 0.0), axis=0, keepdims=True) * (1.0 / N)
  g8 = jnp.broadcast_to(g, (8, D))
  x = jnp.maximum(jnp.dot(g8, w0_ref[...], preferred_element_type=jnp.float32,
                          precision=lax.Precision.HIGHEST) + b0_ref[...], 0.0)
  x = jnp.maximum(jnp.dot(x, w1_ref[...], preferred_element_type=jnp.float32,
                          precision=lax.Precision.HIGHEST) + b1_ref[...], 0.0)
  y = jnp.dot(x, w2_ref[...], preferred_element_type=jnp.float32,
              precision=lax.Precision.HIGHEST) + b2_ref[...]
  o_ref[...] = y


def _readout_call(h, w0, b0, w1, b1, w2, b2):
  return pl.pallas_call(
      _readout_body,
      out_shape=jax.ShapeDtypeStruct((8, D), jnp.float32),
  )(h, w0, b0, w1, b1, w2, b2)


def kernel(node_feat, edge_index, edge_feat, snorm_n, snorm_e, params):
  del edge_feat, snorm_n, snorm_e
  f32 = jnp.float32

  h = jnp.pad(node_feat, ((0, R - N), (0, 0)))
  src = edge_index[0]
  dst = edge_index[1]
  srcs = jnp.concatenate(
      [src, jnp.zeros((EP - E,), jnp.int32)]).reshape(NS * CH, K)
  dsts = jnp.concatenate(
      [dst, jnp.full((EP - E,), DUMP, jnp.int32)]).reshape(NS * CH, K)
  zrows = jnp.zeros((RPT, D), f32)
  ones128 = jnp.ones((K, D), f32)

  segsum = _make_segsum()
  deg_kernel = _make_deg()

  degp = deg_kernel(dsts, ones128, zrows)
  deg128 = degp[:R] + degp[R:]

  for i in range(3):
    lp = params["layers"][i]
    bp = params["bns"][i]
    s = (bp["g"] / jnp.sqrt(bp["rv"] + 1e-5)).reshape(1, D)
    t = (bp["bta"] - bp["rm"] * s[0]).reshape(1, D)
    part = segsum(h, srcs, dsts, zrows)
    h = _mlp_call(h, part.reshape(NC, R, D), deg128,
                  lp["W1"], lp["b1"].reshape(1, H2),
                  lp["W2"], lp["b2"].reshape(1, D), s, t)

  ro = params["readout"]
  w0 = jnp.zeros((D, D), f32).at[:, :64].set(ro[0]["W"])
  b0 = jnp.zeros((1, D), f32).at[0, :64].set(ro[0]["b"])
  w1 = jnp.zeros((D, D), f32).at[:64, :32].set(ro[1]["W"])
  b1 = jnp.zeros((1, D), f32).at[0, :32].set(ro[1]["b"])
  w2 = jnp.zeros((D, D), f32).at[:32, :N_CLASSES].set(ro[2]["W"])
  b2 = jnp.zeros((1, D), f32).at[0, :N_CLASSES].set(ro[2]["b"])
  out = _readout_call(h, w0, b0, w1, b1, w2, b2)
  return out[:1, :N_CLASSES]


# speedup vs baseline: 1.4334x; 1.0373x over previous
import functools

import jax
import jax.numpy as jnp
from jax import lax
from jax.experimental import pallas as pl
from jax.experimental.pallas import tpu as pltpu
from jax.experimental.pallas import tpu_sc as plsc

N = 10000
D = 128
H2 = 256
E = 320000
N_CLASSES = 10

NC = 2
NS = 16
NW = NC * NS
K = 128
CH_A = 120
CH_B = 40
CH = CH_A + CH_B
WIN = 40
NBUF = 2
EP = NS * CH * K
R = 10112
DUMP = N
RPT = R // NS
DEGW = 16
BLK = 1264
GRID = R // BLK


def _mesh():
  return plsc.VectorSubcoreMesh(core_axis_name="c", subcore_axis_name="s",
                                num_cores=NC, num_subcores=NS)


def _segsum_body(h, srcs, dsts, zrows, out, src_v, dst_v,
                 rows0, rows1, acc, s0, s1):
  cid = lax.axis_index("c")
  sid = lax.axis_index("s")
  base = sid * RPT
  rows = (rows0, rows1)
  sems = (s0, s1)

  pltpu.sync_copy(zrows, acc.at[pl.ds(base, RPT)])
  plsc.subcore_barrier()

  def run(row_base, nch):
    for w in range(nch // WIN):
      pltpu.sync_copy(srcs.at[pl.ds(row_base + w * WIN, WIN)], src_v)
      pltpu.sync_copy(dsts.at[pl.ds(row_base + w * WIN, WIN)], dst_v)
      for b in range(NBUF):
        pltpu.async_copy(h.at[src_v.at[b]], rows[b], sems[b])

      def step(i, carry):
        for b in range(NBUF):
          c = i * NBUF + b
          pltpu.make_async_copy(h.at[src_v.at[c]], rows[b], sems[b]).wait()
          pltpu.sync_copy(rows[b], acc.at[dst_v.at[c]], add=True)

          @pl.when(i < WIN // NBUF - 1)
          def _():
            pltpu.async_copy(h.at[src_v.at[c + NBUF]], rows[b], sems[b])
        return carry

      lax.fori_loop(0, WIN // NBUF, step, 0)

  @pl.when(cid == 0)
  def _():
    run(sid * CH_A, CH_A)

  @pl.when(cid == 1)
  def _():
    run(NS * CH_A + sid * CH_B, CH_B)

  plsc.subcore_barrier()

  pltpu.sync_copy(acc.at[pl.ds(base, RPT)],
                  out.at[pl.ds(cid * R + base, RPT)])


def _make_segsum():
  scratch = (
      pltpu.VMEM((WIN, K), jnp.int32),
      pltpu.VMEM((WIN, K), jnp.int32),
      pltpu.VMEM((K, D), jnp.float32),
      pltpu.VMEM((K, D), jnp.float32),
      pltpu.VMEM_SHARED((R, D), jnp.float32),
      pltpu.SemaphoreType.DMA,
      pltpu.SemaphoreType.DMA,
  )
  return pl.kernel(_segsum_body,
                   out_type=jax.ShapeDtypeStruct((NC * R, D), jnp.float32),
                   mesh=_mesh(), scratch_types=scratch)


def _deg_body(dsts, ones128, zrows, degout, dst_v, ones_v, dacc):
  cid = lax.axis_index("c")
  sid = lax.axis_index("s")
  base = sid * RPT

  pltpu.sync_copy(zrows, dacc.at[pl.ds(base, RPT)])
  pltpu.sync_copy(ones128, ones_v)
  plsc.subcore_barrier()

  def run(row_base, nch):
    pltpu.sync_copy(dsts.at[pl.ds(row_base, nch)], dst_v.at[pl.ds(0, nch)])

    def step(c, carry):
      pltpu.sync_copy(ones_v, dacc.at[dst_v.at[c]], add=True)
      return carry

    lax.fori_loop(0, nch, step, 0)

  @pl.when(cid == 0)
  def _():
    run(sid * CH_A, CH_A)

  @pl.when(cid == 1)
  def _():
    run(NS * CH_A + sid * CH_B, CH_B)

  plsc.subcore_barrier()
  pltpu.sync_copy(dacc.at[pl.ds(base, RPT)],
                  degout.at[pl.ds(cid * R + base, RPT)])


def _make_deg():
  scratch = (
      pltpu.VMEM((max(CH_A, CH_B), K), jnp.int32),
      pltpu.VMEM((K, D), jnp.float32),
      pltpu.VMEM_SHARED((R, D), jnp.float32),
  )
  return pl.kernel(_deg_body,
                   out_type=jax.ShapeDtypeStruct((NC * R, D), jnp.float32),
                   mesh=_mesh(), scratch_types=scratch)


def _mlp_body(h_ref, p_ref, dg_ref, w1_ref, b1_ref, w2_ref, b2_ref,
              s_ref, t_ref, o_ref):
  h = h_ref[...]
  agg = (p_ref[0] + p_ref[1]) / jnp.maximum(dg_ref[...], 1.0)
  m = h + agg
  x = jnp.dot(m, w1_ref[...], preferred_element_type=jnp.float32,
              precision=lax.Precision.HIGHEST)
  x = jnp.maximum(x + b1_ref[...], 0.0)
  y = jnp.dot(x, w2_ref[...], preferred_element_type=jnp.float32,
              precision=lax.Precision.HIGHEST)
  y = y + b2_ref[...] + h
  o_ref[...] = y * s_ref[...] + t_ref[...]


def _mlp_call(h, p3, deg128, w1, b1, w2, b2, s, t):
  return pl.pallas_call(
      _mlp_body,
      grid=(GRID,),
      in_specs=[
          pl.BlockSpec((BLK, D), lambda j: (j, 0)),
          pl.BlockSpec((2, BLK, D), lambda j: (0, j, 0)),
          pl.BlockSpec((BLK, D), lambda j: (j, 0)),
          pl.BlockSpec((D, H2), lambda j: (0, 0)),
          pl.BlockSpec((1, H2), lambda j: (0, 0)),
          pl.BlockSpec((H2, D), lambda j: (0, 0)),
          pl.BlockSpec((1, D), lambda j: (0, 0)),
          pl.BlockSpec((1, D), lambda j: (0, 0)),
          pl.BlockSpec((1, D), lambda j: (0, 0)),
      ],
      out_specs=pl.BlockSpec((BLK, D), lambda j: (j, 0)),
      out_shape=jax.ShapeDtypeStruct((R, D), jnp.float32),
  )(h, p3, deg128, w1, b1, w2, b2, s, t)


def _readout_body(h_ref, w0_ref, b0_ref, w1_ref, b1_ref, w2_ref, b2_ref,
                  o_ref):
  h = h_ref[...]
  rows = lax.broadcasted_iota(jnp.int32, (R, 1), 0)
  g = jnp.sum(jnp.where(rows < N, h, 0.0), axis=0, keepdims=True) * (1.0 / N)
  g8 = jnp.broadcast_to(g, (8, D))
  x = jnp.maximum(jnp.dot(g8, w0_ref[...], preferred_element_type=jnp.float32,
                          precision=lax.Precision.HIGHEST) + b0_ref[...], 0.0)
  x = jnp.maximum(jnp.dot(x, w1_ref[...], preferred_element_type=jnp.float32,
                          precision=lax.Precision.HIGHEST) + b1_ref[...], 0.0)
  y = jnp.dot(x, w2_ref[...], preferred_element_type=jnp.float32,
              precision=lax.Precision.HIGHEST) + b2_ref[...]
  o_ref[...] = y


def _readout_call(h, w0, b0, w1, b1, w2, b2):
  return pl.pallas_call(
      _readout_body,
      out_shape=jax.ShapeDtypeStruct((8, D), jnp.float32),
  )(h, w0, b0, w1, b1, w2, b2)


def kernel(node_feat, edge_index, edge_feat, snorm_n, snorm_e, params):
  del edge_feat, snorm_n, snorm_e
  f32 = jnp.float32

  h = jnp.pad(node_feat, ((0, R - N), (0, 0)))
  src = edge_index[0]
  dst = edge_index[1]
  srcs = jnp.concatenate(
      [src, jnp.zeros((EP - E,), jnp.int32)]).reshape(NS * CH, K)
  dsts = jnp.concatenate(
      [dst, jnp.full((EP - E,), DUMP, jnp.int32)]).reshape(NS * CH, K)
  zrows = jnp.zeros((RPT, D), f32)
  ones128 = jnp.ones((K, D), f32)

  segsum = _make_segsum()
  deg_kernel = _make_deg()

  degp = deg_kernel(dsts, ones128, zrows)
  deg128 = degp[:R] + degp[R:]

  for i in range(3):
    lp = params["layers"][i]
    bp = params["bns"][i]
    s = (bp["g"] / jnp.sqrt(bp["rv"] + 1e-5)).reshape(1, D)
    t = (bp["bta"] - bp["rm"] * s[0]).reshape(1, D)
    part = segsum(h, srcs, dsts, zrows)
    h = _mlp_call(h, part.reshape(NC, R, D), deg128,
                  lp["W1"], lp["b1"].reshape(1, H2),
                  lp["W2"], lp["b2"].reshape(1, D), s, t)

  ro = params["readout"]
  w0 = jnp.zeros((D, D), f32).at[:, :64].set(ro[0]["W"])
  b0 = jnp.zeros((1, D), f32).at[0, :64].set(ro[0]["b"])
  w1 = jnp.zeros((D, D), f32).at[:64, :32].set(ro[1]["W"])
  b1 = jnp.zeros((1, D), f32).at[0, :32].set(ro[1]["b"])
  w2 = jnp.zeros((D, D), f32).at[:32, :N_CLASSES].set(ro[2]["W"])
  b2 = jnp.zeros((1, D), f32).at[0, :N_CLASSES].set(ro[2]["b"])
  out = _readout_call(h, w0, b0, w1, b1, w2, b2)
  return out[:1, :N_CLASSES]
